# speedup vs baseline: 4.3434x; 4.3434x over previous
import functools

import jax
import jax.numpy as jnp
from jax import lax
from jax.experimental import pallas as pl
from jax.experimental.pallas import tpu as pltpu
from jax.experimental.pallas import tpu_sc as plsc

N = 10000
E = 320000
D = 128
H = 128
C = 10

NC = 2
NS = 16
NW = NC * NS
NP = 10240
RPT = NP // NS
EPW = E // NW
K = 80
NCHUNK = EPW // K

_sc_mesh = plsc.VectorSubcoreMesh(core_axis_name="c", subcore_axis_name="s")


@functools.partial(
    pl.kernel,
    out_type=jax.ShapeDtypeStruct((NC, 2, NP), jnp.float32),
    mesh=_sc_mesh,
    scratch_types=[
        pltpu.VMEM((K,), jnp.int32),
        pltpu.VMEM((K,), jnp.float32),
        pltpu.VMEM_SHARED((NP,), jnp.float32),
        pltpu.VMEM_SHARED((NP,), jnp.float32),
    ],
)
def _deg_kernel(src_hbm, dst_hbm, zrow_hbm, out_hbm, idx_v, ones_v, acc_o, acc_i):
    c = lax.axis_index("c")
    s = lax.axis_index("s")
    wid = s * NC + c

    for j in range(K // 16):
        ones_v[pl.ds(j * 16, 16)] = jnp.ones((16,), jnp.float32)

    pltpu.sync_copy(zrow_hbm, acc_o.at[pl.ds(s * RPT, RPT)])
    pltpu.sync_copy(zrow_hbm, acc_i.at[pl.ds(s * RPT, RPT)])
    plsc.subcore_barrier()

    def body(i, _):
        base = wid * EPW + i * K
        pltpu.sync_copy(src_hbm.at[pl.ds(base, K)], idx_v)
        pltpu.sync_copy(ones_v, acc_o.at[idx_v], add=True)
        pltpu.sync_copy(dst_hbm.at[pl.ds(base, K)], idx_v)
        pltpu.sync_copy(ones_v, acc_i.at[idx_v], add=True)
        return 0

    lax.fori_loop(0, NCHUNK, body, 0)
    plsc.subcore_barrier()

    pltpu.sync_copy(acc_o.at[pl.ds(s * RPT, RPT)], out_hbm.at[c, 0, pl.ds(s * RPT, RPT)])
    pltpu.sync_copy(acc_i.at[pl.ds(s * RPT, RPT)], out_hbm.at[c, 1, pl.ds(s * RPT, RPT)])


@functools.partial(
    pl.kernel,
    out_type=jax.ShapeDtypeStruct((NC, NP, H), jnp.float32),
    mesh=_sc_mesh,
    scratch_types=[
        pltpu.VMEM((K,), jnp.int32),
        pltpu.VMEM((K,), jnp.int32),
        pltpu.VMEM((K, H), jnp.float32),
        pltpu.VMEM_SHARED((NP, H), jnp.float32),
        pltpu.SemaphoreType.DMA,
    ],
)
def _agg_kernel(t_hbm, src_hbm, dst_hbm, zrows_hbm, out_hbm, srcv, dstv, rows, acc, sem):
    c = lax.axis_index("c")
    s = lax.axis_index("s")
    wid = s * NC + c

    pltpu.sync_copy(zrows_hbm, acc.at[pl.ds(s * RPT, RPT)])
    plsc.subcore_barrier()

    def body(i, _):
        base = wid * EPW + i * K
        pltpu.sync_copy(src_hbm.at[pl.ds(base, K)], srcv)
        pltpu.sync_copy(dst_hbm.at[pl.ds(base, K)], dstv)
        pltpu.async_copy(t_hbm.at[srcv], rows, sem).wait()
        pltpu.sync_copy(rows, acc.at[dstv], add=True)
        return 0

    lax.fori_loop(0, NCHUNK, body, 0)
    plsc.subcore_barrier()

    pltpu.sync_copy(acc.at[pl.ds(s * RPT, RPT)], out_hbm.at[c, pl.ds(s * RPT, RPT)])


R = 2000
GRID = N // R


def _norms(deg_blk):
    ns = lax.rsqrt(jnp.clip(deg_blk[0, 0] + deg_blk[1, 0], 1.0, None))
    nd = lax.rsqrt(jnp.clip(deg_blk[0, 1] + deg_blk[1, 1], 1.0, None))
    return ns, nd


def _tc_in_body(x_ref, wfc_ref, bfc_ref, deg_ref, w1_ref, o_ref):
    h1 = jnp.maximum(jnp.dot(x_ref[...], wfc_ref[...],
                             preferred_element_type=jnp.float32) + bfc_ref[...], 0.0)
    ns, _ = _norms(deg_ref[...])
    o_ref[...] = jnp.dot(h1 * ns, w1_ref[...], preferred_element_type=jnp.float32)


def _tc_in(x, W_fc, b_fc2, degp, W1):
    return pl.pallas_call(
        _tc_in_body,
        grid=(GRID,),
        in_specs=[
            pl.BlockSpec((R, D), lambda i: (i, 0)),
            pl.BlockSpec((D, H), lambda i: (0, 0)),
            pl.BlockSpec((1, H), lambda i: (0, 0)),
            pl.BlockSpec((NC, 2, R, 1), lambda i: (0, 0, i, 0)),
            pl.BlockSpec((H, H), lambda i: (0, 0)),
        ],
        out_specs=pl.BlockSpec((R, H), lambda i: (i, 0)),
        out_shape=jax.ShapeDtypeStruct((N, H), jnp.float32),
    )(x, W_fc, b_fc2, degp, W1)


def _tc_mid_body(p_ref, deg_ref, b_ref, w_ref, o_ref):
    agg = p_ref[0] + p_ref[1]
    ns, nd = _norms(deg_ref[...])
    h = jnp.maximum(agg * nd + b_ref[...], 0.0)
    o_ref[...] = jnp.dot(h * ns, w_ref[...], preferred_element_type=jnp.float32)


def _tc_mid(p, degp, b2, W):
    return pl.pallas_call(
        _tc_mid_body,
        grid=(GRID,),
        in_specs=[
            pl.BlockSpec((NC, R, H), lambda i: (0, i, 0)),
            pl.BlockSpec((NC, 2, R, 1), lambda i: (0, 0, i, 0)),
            pl.BlockSpec((1, H), lambda i: (0, 0)),
            pl.BlockSpec((H, H), lambda i: (0, 0)),
        ],
        out_specs=pl.BlockSpec((R, H), lambda i: (i, 0)),
        out_shape=jax.ShapeDtypeStruct((N, H), jnp.float32),
    )(p, degp, b2, W)


def _tc_out_body(p_ref, deg_ref, b_ref, wc_ref, bc_ref, o_ref):
    agg = p_ref[0] + p_ref[1]
    nd = lax.rsqrt(jnp.clip(deg_ref[0, 1] + deg_ref[1, 1], 1.0, None))
    h4 = jnp.maximum(agg * nd + b_ref[...], 0.0)
    rep = jnp.sum(h4, axis=0, keepdims=True) * (1.0 / N)
    o_ref[...] = jnp.dot(rep, wc_ref[...], preferred_element_type=jnp.float32) + bc_ref[...]


def _tc_out(p, degp, b2, Wc, bc2):
    return pl.pallas_call(
        _tc_out_body,
        grid=(1,),
        in_specs=[
            pl.BlockSpec((NC, N, H), lambda i: (0, 0, 0)),
            pl.BlockSpec((NC, 2, N, 1), lambda i: (0, 0, 0, 0)),
            pl.BlockSpec((1, H), lambda i: (0, 0)),
            pl.BlockSpec((H, C), lambda i: (0, 0)),
            pl.BlockSpec((1, C), lambda i: (0, 0)),
        ],
        out_specs=pl.BlockSpec((1, C), lambda i: (0, 0)),
        out_shape=jax.ShapeDtypeStruct((1, C), jnp.float32),
    )(p, degp, b2, Wc, bc2)


def kernel(inputs, edge_index, W_fc, b_fc, W1, b1, W2, b2, W3, b3, Wc, bc):
    src = edge_index[0].astype(jnp.int32)
    dst = edge_index[1].astype(jnp.int32)
    zrow = jnp.zeros((RPT,), jnp.float32)
    zrows = jnp.zeros((RPT, H), jnp.float32)

    degp = _deg_kernel(src, dst, zrow)
    degp = degp[:, :, :, None]

    t1 = _tc_in(inputs, W_fc, b_fc[None, :], degp, W1)
    p1 = _agg_kernel(t1, src, dst, zrows)
    t2 = _tc_mid(p1, degp, b1[None, :], W2)
    p2 = _agg_kernel(t2, src, dst, zrows)
    t3 = _tc_mid(p2, degp, b2[None, :], W3)
    p3 = _agg_kernel(t3, src, dst, zrows)
    return _tc_out(p3, degp, b3[None, :], Wc, bc[None, :])


# speedup vs baseline: 10.5666x; 2.4328x over previous
import functools

import jax
import jax.numpy as jnp
from jax import lax
from jax.experimental import pallas as pl
from jax.experimental.pallas import tpu as pltpu
from jax.experimental.pallas import tpu_sc as plsc

N = 10000
E = 320000
D = 128
H = 128
C = 10

NC = 2
NS = 16
NW = NC * NS
NP = 10240
RPT = NP // NS
EPW = E // NW
K = 80
NCHUNK = EPW // K

_sc_mesh = plsc.VectorSubcoreMesh(core_axis_name="c", subcore_axis_name="s")


NBUF = 5
NGRP = NCHUNK // NBUF


def _fill_idx(buf, slab, e):
    for i in range(K // 16):
        buf[pl.ds(i * 16, 16)] = slab[pl.ds(e * K + i * 16, 16)]


@functools.partial(
    pl.kernel,
    out_type=jax.ShapeDtypeStruct((NC, 2, NP), jnp.float32),
    mesh=_sc_mesh,
    scratch_types=[
        pltpu.VMEM((EPW,), jnp.int32),
        pltpu.VMEM((EPW,), jnp.int32),
        pltpu.VMEM((K,), jnp.float32),
        pltpu.VMEM_SHARED((NP,), jnp.float32),
        pltpu.VMEM_SHARED((NP,), jnp.float32),
    ] + [pltpu.VMEM((K,), jnp.int32)] * NBUF
      + [pltpu.SemaphoreType.DMA] * NBUF,
)
def _deg_kernel(src_hbm, dst_hbm, zrow_hbm, out_hbm, srcs, dsts, ones_v, acc_o, acc_i, *rest):
    idxb = rest[:NBUF]
    sems = rest[NBUF:]
    c = lax.axis_index("c")
    s = lax.axis_index("s")
    wid = s * NC + c

    for j in range(K // 16):
        ones_v[pl.ds(j * 16, 16)] = jnp.ones((16,), jnp.float32)

    pltpu.sync_copy(src_hbm.at[wid], srcs)
    pltpu.sync_copy(dst_hbm.at[wid], dsts)

    pltpu.sync_copy(zrow_hbm, acc_o.at[pl.ds(s * RPT, RPT)])
    pltpu.sync_copy(zrow_hbm, acc_i.at[pl.ds(s * RPT, RPT)])
    plsc.subcore_barrier()

    def mkbody(slab, acc):
        def body(j, _):
            i0 = j * NBUF
            d = []
            for b in range(NBUF):
                _fill_idx(idxb[b], slab, i0 + b)
                d.append(pltpu.async_copy(ones_v, acc.at[idxb[b]], sems[b], add=True))
            for x in d:
                x.wait()
            return 0
        return body

    lax.fori_loop(0, NGRP, mkbody(srcs, acc_o), 0)
    lax.fori_loop(0, NGRP, mkbody(dsts, acc_i), 0)
    plsc.subcore_barrier()

    pltpu.sync_copy(acc_o.at[pl.ds(s * RPT, RPT)], out_hbm.at[c, 0, pl.ds(s * RPT, RPT)])
    pltpu.sync_copy(acc_i.at[pl.ds(s * RPT, RPT)], out_hbm.at[c, 1, pl.ds(s * RPT, RPT)])


@functools.partial(
    pl.kernel,
    out_type=jax.ShapeDtypeStruct((NC, NP, H), jnp.float32),
    mesh=_sc_mesh,
    scratch_types=[
        pltpu.VMEM((EPW,), jnp.int32),
        pltpu.VMEM((EPW,), jnp.int32),
        pltpu.VMEM_SHARED((NP, H), jnp.float32),
        pltpu.VMEM((K, H), jnp.float32),
        pltpu.VMEM((K, H), jnp.float32),
        pltpu.VMEM((K,), jnp.int32),
        pltpu.VMEM((K,), jnp.int32),
        pltpu.VMEM((K,), jnp.int32),
        pltpu.VMEM((K,), jnp.int32),
        pltpu.SemaphoreType.DMA,
        pltpu.SemaphoreType.DMA,
        pltpu.SemaphoreType.DMA,
        pltpu.SemaphoreType.DMA,
    ],
)
def _agg_kernel(t_hbm, src_hbm, dst_hbm, zrows_hbm, out_hbm, srcs, dsts, acc,
                rows0, rows1, sb0, sb1, db0, db1, gsem0, gsem1, ssem0, ssem1):
    rows = (rows0, rows1)
    sbuf = (sb0, sb1)
    dbuf = (db0, db1)
    gsem = (gsem0, gsem1)
    ssem = (ssem0, ssem1)
    c = lax.axis_index("c")
    s = lax.axis_index("s")
    wid = s * NC + c

    pltpu.sync_copy(src_hbm.at[wid], srcs)
    pltpu.sync_copy(dst_hbm.at[wid], dsts)

    pltpu.sync_copy(zrows_hbm, acc.at[pl.ds(s * RPT, RPT)])
    plsc.subcore_barrier()

    def wait_g(b):
        pltpu.make_async_copy(t_hbm.at[sbuf[b]], rows[b], gsem[b]).wait()

    def wait_s(b):
        pltpu.make_async_copy(rows[b], acc.at[dbuf[b]], ssem[b]).wait()

    def issue_g(b, e):
        _fill_idx(sbuf[b], srcs, e)
        pltpu.async_copy(t_hbm.at[sbuf[b]], rows[b], gsem[b])

    def issue_s(b, e):
        _fill_idx(dbuf[b], dsts, e)
        pltpu.async_copy(rows[b], acc.at[dbuf[b]], ssem[b], add=True)

    issue_g(0, 0)
    issue_g(1, 1)

    def body(j, _):
        for b in range(2):
            e = 2 * j + b
            wait_g(b)
            issue_s(b, e)
            wait_s(b)

            @pl.when(e + 2 < NCHUNK)
            def _():
                issue_g(b, e + 2)
        return 0

    lax.fori_loop(0, NCHUNK // 2, body, 0)
    wait_g(0)
    issue_s(0, NCHUNK - 1)
    wait_s(0)

    plsc.subcore_barrier()
    pltpu.sync_copy(acc.at[pl.ds(s * RPT, RPT)], out_hbm.at[c, pl.ds(s * RPT, RPT)])


R = 2000
GRID = N // R


def _norms(deg_blk):
    ns = lax.rsqrt(jnp.clip(deg_blk[0, 0] + deg_blk[1, 0], 1.0, None))
    nd = lax.rsqrt(jnp.clip(deg_blk[0, 1] + deg_blk[1, 1], 1.0, None))
    return ns, nd


def _tc_in_body(x_ref, wfc_ref, bfc_ref, deg_ref, w1_ref, o_ref):
    h1 = jnp.maximum(jnp.dot(x_ref[...], wfc_ref[...],
                             preferred_element_type=jnp.float32) + bfc_ref[...], 0.0)
    ns, _ = _norms(deg_ref[...])
    o_ref[...] = jnp.dot(h1 * ns, w1_ref[...], preferred_element_type=jnp.float32)


def _tc_in(x, W_fc, b_fc2, degp, W1):
    return pl.pallas_call(
        _tc_in_body,
        grid=(GRID,),
        in_specs=[
            pl.BlockSpec((R, D), lambda i: (i, 0)),
            pl.BlockSpec((D, H), lambda i: (0, 0)),
            pl.BlockSpec((1, H), lambda i: (0, 0)),
            pl.BlockSpec((NC, 2, R, 1), lambda i: (0, 0, i, 0)),
            pl.BlockSpec((H, H), lambda i: (0, 0)),
        ],
        out_specs=pl.BlockSpec((R, H), lambda i: (i, 0)),
        out_shape=jax.ShapeDtypeStruct((N, H), jnp.float32),
    )(x, W_fc, b_fc2, degp, W1)


def _tc_mid_body(p_ref, deg_ref, b_ref, w_ref, o_ref):
    agg = p_ref[0] + p_ref[1]
    ns, nd = _norms(deg_ref[...])
    h = jnp.maximum(agg * nd + b_ref[...], 0.0)
    o_ref[...] = jnp.dot(h * ns, w_ref[...], preferred_element_type=jnp.float32)


def _tc_mid(p, degp, b2, W):
    return pl.pallas_call(
        _tc_mid_body,
        grid=(GRID,),
        in_specs=[
            pl.BlockSpec((NC, R, H), lambda i: (0, i, 0)),
            pl.BlockSpec((NC, 2, R, 1), lambda i: (0, 0, i, 0)),
            pl.BlockSpec((1, H), lambda i: (0, 0)),
            pl.BlockSpec((H, H), lambda i: (0, 0)),
        ],
        out_specs=pl.BlockSpec((R, H), lambda i: (i, 0)),
        out_shape=jax.ShapeDtypeStruct((N, H), jnp.float32),
    )(p, degp, b2, W)


def _tc_out_body(p_ref, deg_ref, b_ref, wc_ref, bc_ref, o_ref):
    agg = p_ref[0] + p_ref[1]
    nd = lax.rsqrt(jnp.clip(deg_ref[0, 1] + deg_ref[1, 1], 1.0, None))
    h4 = jnp.maximum(agg * nd + b_ref[...], 0.0)
    rep = jnp.sum(h4, axis=0, keepdims=True) * (1.0 / N)
    o_ref[...] = jnp.dot(rep, wc_ref[...], preferred_element_type=jnp.float32) + bc_ref[...]


def _tc_out(p, degp, b2, Wc, bc2):
    return pl.pallas_call(
        _tc_out_body,
        grid=(1,),
        in_specs=[
            pl.BlockSpec((NC, N, H), lambda i: (0, 0, 0)),
            pl.BlockSpec((NC, 2, N, 1), lambda i: (0, 0, 0, 0)),
            pl.BlockSpec((1, H), lambda i: (0, 0)),
            pl.BlockSpec((H, C), lambda i: (0, 0)),
            pl.BlockSpec((1, C), lambda i: (0, 0)),
        ],
        out_specs=pl.BlockSpec((1, C), lambda i: (0, 0)),
        out_shape=jax.ShapeDtypeStruct((1, C), jnp.float32),
    )(p, degp, b2, Wc, bc2)


def kernel(inputs, edge_index, W_fc, b_fc, W1, b1, W2, b2, W3, b3, Wc, bc):
    src = edge_index[0].astype(jnp.int32).reshape(NW, EPW)
    dst = edge_index[1].astype(jnp.int32).reshape(NW, EPW)
    zrow = jnp.zeros((RPT,), jnp.float32)
    zrows = jnp.zeros((RPT, H), jnp.float32)

    degp = _deg_kernel(src, dst, zrow)
    degp = degp[:, :, :, None]

    t1 = _tc_in(inputs, W_fc, b_fc[None, :], degp, W1)
    p1 = _agg_kernel(t1, src, dst, zrows)
    t2 = _tc_mid(p1, degp, b1[None, :], W2)
    p2 = _agg_kernel(t2, src, dst, zrows)
    t3 = _tc_mid(p2, degp, b2[None, :], W3)
    p3 = _agg_kernel(t3, src, dst, zrows)
    return _tc_out(p3, degp, b3[None, :], Wc, bc[None, :])
